# Initial kernel scaffold; baseline (speedup 1.0000x reference)
import jax
import jax.numpy as jnp
from jax import lax
from jax.experimental import pallas as pl
from jax.experimental.pallas import tpu as pltpu
from jax.experimental.pallas import tpu_sc as plsc

N = 10000
E = 320000
D = 128
H = 4
DH = 32
NLAYER = 4
NC = 2
NS = 16
NW = NC * NS
EW = E // NW
CH = 80
NCH = EW // CH
RPT = N // NS
BN = 1000
NB = N // BN

F32 = jnp.float32
I32 = jnp.int32


def _sc_mesh():
    return plsc.VectorSubcoreMesh(core_axis_name="c", subcore_axis_name="s",
                                  num_cores=NC, num_subcores=NS)


def _mm0_body(x_ref, w_ref, hs_ref, hn0_ref, hn1_ref):
    acc = jnp.dot(x_ref[...], w_ref[...], preferred_element_type=F32)
    hs_ref[...] = acc[:, :D]
    hn0_ref[...] = acc[:, D:2 * D]
    hn1_ref[...] = acc[:, 2 * D:]


def _combine(hs_ref, a00, a01, a10, a11, d00, d01, d10, d11, b_ref):
    agg0 = a00[...] + a01[...]
    agg1 = a10[...] + a11[...]
    deg0 = jnp.maximum(d00[:, 0:1] + d01[:, 0:1], 1.0)
    deg1 = jnp.maximum(d10[:, 0:1] + d11[:, 0:1], 1.0)
    h = 0.5 * (hs_ref[...] + agg0 / deg0 + agg1 / deg1 + b_ref[...])
    return jnp.maximum(h, 0.01 * h)


def _layer_body(hs_ref, a00, a01, a10, a11, d00, d01, d10, d11, b_ref, w_ref,
                hs_o, hn0_o, hn1_o):
    h = _combine(hs_ref, a00, a01, a10, a11, d00, d01, d10, d11, b_ref)
    acc = jnp.dot(h, w_ref[...], preferred_element_type=F32)
    hs_o[...] = acc[:, :D]
    hn0_o[...] = acc[:, D:2 * D]
    hn1_o[...] = acc[:, 2 * D:]


def _gat_head_body(hs_ref, a00, a01, a10, a11, d00, d01, d10, d11, b_ref,
                   w_ref, al0_ref, al1_ref, ar0_ref, ar1_ref, g_ref,
                   f0_o, f1_o, elp0_o, elp1_o, erp0_o, erp1_o,
                   mel0_o, mel1_o, mer0_o, mer1_o):
    h = _combine(hs_ref, a00, a01, a10, a11, d00, d01, d10, d11, b_ref)
    acc = jnp.dot(h, w_ref[...], preferred_element_type=F32)
    f0 = acc[:, :D]
    f1 = acc[:, D:]
    f0_o[...] = f0
    f1_o[...] = f1
    g = g_ref[...]
    elp0 = jnp.dot(f0 * al0_ref[...], g, preferred_element_type=F32)
    elp1 = jnp.dot(f1 * al1_ref[...], g, preferred_element_type=F32)
    erp0 = jnp.dot(f0 * ar0_ref[...], g, preferred_element_type=F32)
    erp1 = jnp.dot(f1 * ar1_ref[...], g, preferred_element_type=F32)
    elp0_o[...] = elp0
    elp1_o[...] = elp1
    erp0_o[...] = erp0
    erp1_o[...] = erp1
    i = pl.program_id(0)

    @pl.when(i == 0)
    def _():
        mel0_o[...] = jnp.full_like(mel0_o, -1e30)
        mel1_o[...] = jnp.full_like(mel1_o, -1e30)
        mer0_o[...] = jnp.full_like(mer0_o, -1e30)
        mer1_o[...] = jnp.full_like(mer1_o, -1e30)

    mel0_o[...] = jnp.maximum(mel0_o[...], jnp.max(elp0))
    mel1_o[...] = jnp.maximum(mel1_o[...], jnp.max(elp1))
    mer0_o[...] = jnp.maximum(mer0_o[...], jnp.max(erp0))
    mer1_o[...] = jnp.maximum(mer1_o[...], jnp.max(erp1))


def _final_body(n00, n01, n10, n11, d00, d01, d10, d11, gt_ref, w_ref, b_ref,
                o_ref):
    gt = gt_ref[...]
    den0 = jnp.dot(d00[...] + d01[...], gt, preferred_element_type=F32)
    den1 = jnp.dot(d10[...] + d11[...], gt, preferred_element_type=F32)
    g0 = (n00[...] + n01[...]) / jnp.maximum(den0, 1e-30)
    g1 = (n10[...] + n11[...]) / jnp.maximum(den1, 1e-30)
    g = 0.5 * (g0 + g1)
    o_ref[...] = jnp.dot(g, w_ref[...], preferred_element_type=F32) + b_ref[...]


def _row_spec(k=0):
    return pl.BlockSpec((BN, D), lambda i, k=k: (k * NB + i, 0))


def _deg_spec(k):
    return pl.BlockSpec((BN, 16), lambda i, k=k: (k * NB + i, 0))


def _full_spec(shape):
    return pl.BlockSpec(shape, lambda i: tuple(0 for _ in shape))


def _spmm_body(hn_ref, src_ref, dst_ref, zer_ref, out_ref,
               acc, sbuf, dbuf, rows, sem):
    c = lax.axis_index("c")
    s = lax.axis_index("s")
    wid = c * NS + s
    pltpu.sync_copy(zer_ref, acc.at[pl.ds(s * RPT, RPT)])
    plsc.subcore_barrier()
    base = wid * EW

    @pl.loop(0, NCH)
    def _(j):
        off = base + j * CH
        pltpu.sync_copy(src_ref.at[pl.ds(off, CH)], sbuf)
        pltpu.sync_copy(dst_ref.at[pl.ds(off, CH)], dbuf)
        pltpu.async_copy(hn_ref.at[sbuf], rows, sem).wait()
        pltpu.sync_copy(rows, acc.at[dbuf], add=True)

    plsc.subcore_barrier()
    pltpu.sync_copy(acc.at[pl.ds(s * RPT, RPT)],
                    out_ref.at[pl.ds(c * N + s * RPT, RPT)])


def _deg_body(dst0_ref, dst1_ref, zer_ref, ones_ref, out_ref,
              acc0, acc1, dbuf, ones_v):
    c = lax.axis_index("c")
    s = lax.axis_index("s")
    wid = c * NS + s
    pltpu.sync_copy(zer_ref, acc0.at[pl.ds(s * RPT, RPT)])
    pltpu.sync_copy(zer_ref, acc1.at[pl.ds(s * RPT, RPT)])
    pltpu.sync_copy(ones_ref, ones_v)
    plsc.subcore_barrier()
    base = wid * EW

    @pl.loop(0, NCH)
    def _(j):
        off = base + j * CH
        pltpu.sync_copy(dst0_ref.at[pl.ds(off, CH)], dbuf)
        pltpu.sync_copy(ones_v, acc0.at[dbuf], add=True)
        pltpu.sync_copy(dst1_ref.at[pl.ds(off, CH)], dbuf)
        pltpu.sync_copy(ones_v, acc1.at[dbuf], add=True)

    plsc.subcore_barrier()
    pltpu.sync_copy(acc0.at[pl.ds(s * RPT, RPT)],
                    out_ref.at[pl.ds(c * N + s * RPT, RPT)])
    pltpu.sync_copy(acc1.at[pl.ds(s * RPT, RPT)],
                    out_ref.at[pl.ds((2 + c) * N + s * RPT, RPT)])


def _gat_body(feat_ref, elp_ref, erp_ref, mel_ref, mer_ref, src_ref, dst_ref,
              z128_ref, z16_ref, num_out, den_out,
              accn, accd, sbuf, dbuf, fbuf, elb, erb, mv, sem):
    c = lax.axis_index("c")
    s = lax.axis_index("s")
    wid = c * NS + s
    pltpu.sync_copy(z128_ref, accn.at[pl.ds(s * RPT, RPT)])
    pltpu.sync_copy(z16_ref, accd.at[pl.ds(s * RPT, RPT)])
    pltpu.sync_copy(mel_ref, mv)
    pltpu.sync_copy(mer_ref, elb.at[0])
    plsc.subcore_barrier()
    mv[...] = mv[...] + elb[0, :]
    lane = lax.broadcasted_iota(I32, (16,), 0)
    base = wid * EW

    @pl.loop(0, NCH)
    def _(j):
        off = base + j * CH
        pltpu.sync_copy(src_ref.at[pl.ds(off, CH)], sbuf)
        pltpu.sync_copy(dst_ref.at[pl.ds(off, CH)], dbuf)
        pltpu.async_copy(elp_ref.at[sbuf], elb, sem).wait()
        pltpu.async_copy(erp_ref.at[dbuf], erb, sem).wait()
        pltpu.async_copy(feat_ref.at[sbuf], fbuf, sem).wait()
        m = mv[...]

        @pl.loop(0, CH)
        def _(i):
            e = elb[i, :] + erb[i, :]
            e = jnp.maximum(e, 0.2 * e)
            ee = jnp.exp(e - m)
            ee = jnp.where(lane < H, ee, 0.0)
            elb[i, :] = ee
            for hh in range(H):
                sp = jnp.take(ee, jnp.full((16,), hh, I32),
                              mode=lax.GatherScatterMode.PROMISE_IN_BOUNDS)
                for q in range(2):
                    col = hh * DH + q * 16
                    fbuf[i, pl.ds(col, 16)] = fbuf[i, pl.ds(col, 16)] * sp

        pltpu.sync_copy(fbuf, accn.at[dbuf], add=True)
        pltpu.sync_copy(elb, accd.at[dbuf], add=True)

    plsc.subcore_barrier()
    pltpu.sync_copy(accn.at[pl.ds(s * RPT, RPT)],
                    num_out.at[pl.ds(c * N + s * RPT, RPT)])
    pltpu.sync_copy(accd.at[pl.ds(s * RPT, RPT)],
                    den_out.at[pl.ds(c * N + s * RPT, RPT)])


def _make_spmm():
    return pl.kernel(
        _spmm_body,
        out_type=jax.ShapeDtypeStruct((2 * N, D), F32),
        mesh=_sc_mesh(),
        scratch_types=[
            pltpu.VMEM_SHARED((N, D), F32),
            pltpu.VMEM((CH,), I32),
            pltpu.VMEM((CH,), I32),
            pltpu.VMEM((CH, D), F32),
            pltpu.SemaphoreType.DMA,
        ],
    )


def _make_deg():
    return pl.kernel(
        _deg_body,
        out_type=jax.ShapeDtypeStruct((4 * N, 16), F32),
        mesh=_sc_mesh(),
        scratch_types=[
            pltpu.VMEM_SHARED((N, 16), F32),
            pltpu.VMEM_SHARED((N, 16), F32),
            pltpu.VMEM((CH,), I32),
            pltpu.VMEM((CH, 16), F32),
        ],
    )


def _make_gat():
    return pl.kernel(
        _gat_body,
        out_type=[jax.ShapeDtypeStruct((2 * N, D), F32),
                  jax.ShapeDtypeStruct((2 * N, 16), F32)],
        mesh=_sc_mesh(),
        scratch_types=[
            pltpu.VMEM_SHARED((N, D), F32),
            pltpu.VMEM_SHARED((N, 16), F32),
            pltpu.VMEM((CH,), I32),
            pltpu.VMEM((CH,), I32),
            pltpu.VMEM((CH, D), F32),
            pltpu.VMEM((CH, 16), F32),
            pltpu.VMEM((CH, 16), F32),
            pltpu.VMEM((16,), F32),
            pltpu.SemaphoreType.DMA,
        ],
    )


def kernel(x, edge_index_r0, edge_index_r1, sage_Wself, sage_Wneigh, sage_b,
           gat_W, gat_attn_l, gat_attn_r, lin_W, lin_b):
    src0, dst0 = edge_index_r0[0], edge_index_r0[1]
    src1, dst1 = edge_index_r1[0], edge_index_r1[1]

    wpack = [jnp.concatenate([sage_Wself[l, 0] + sage_Wself[l, 1],
                              sage_Wneigh[l, 0], sage_Wneigh[l, 1]], axis=1)
             for l in range(NLAYER)]
    bsum = [(sage_b[l, 0] + sage_b[l, 1]).reshape(1, D) for l in range(NLAYER)]
    gwpack = jnp.concatenate([gat_W[0], gat_W[1]], axis=1)
    al0 = gat_attn_l[0].reshape(1, D)
    al1 = gat_attn_l[1].reshape(1, D)
    ar0 = gat_attn_r[0].reshape(1, D)
    ar1 = gat_attn_r[1].reshape(1, D)
    gmat = (jnp.arange(16)[None, :] == (jnp.arange(D)[:, None] // DH)).astype(F32)
    gmat_t = gmat.T
    linb2 = lin_b.reshape(1, D)
    z128 = jnp.zeros((RPT, D), F32)
    z16 = jnp.zeros((RPT, 16), F32)
    ones16 = jnp.ones((CH, 16), F32)

    spmm = _make_spmm()
    gat = _make_gat()

    degp = _make_deg()(dst0, dst1, z16, ones16)

    row_o = pl.BlockSpec((BN, D), lambda i: (i, 0))
    w384 = _full_spec((D, 3 * D))
    b1 = _full_spec((1, D))

    hs, hn0, hn1 = pl.pallas_call(
        _mm0_body,
        grid=(NB,),
        in_specs=[row_o, w384],
        out_specs=[row_o, row_o, row_o],
        out_shape=[jax.ShapeDtypeStruct((N, D), F32)] * 3,
    )(x, wpack[0])

    deg_specs = [_deg_spec(0), _deg_spec(1), _deg_spec(2), _deg_spec(3)]

    f0 = f1 = elp0 = elp1 = erp0 = erp1 = None
    mel0 = mel1 = mer0 = mer1 = None
    for l in range(NLAYER):
        aggp0 = spmm(hn0, src0, dst0, z128)
        aggp1 = spmm(hn1, src1, dst1, z128)
        agg_in = [aggp0, aggp0, aggp1, aggp1, degp, degp, degp, degp]
        agg_specs = [_row_spec(0), _row_spec(1), _row_spec(0), _row_spec(1)] + deg_specs
        if l < NLAYER - 1:
            hs, hn0, hn1 = pl.pallas_call(
                _layer_body,
                grid=(NB,),
                in_specs=[row_o] + agg_specs + [b1, w384],
                out_specs=[row_o, row_o, row_o],
                out_shape=[jax.ShapeDtypeStruct((N, D), F32)] * 3,
            )(hs, *agg_in, bsum[l], wpack[l + 1])
        else:
            m8 = pl.BlockSpec((8, D), lambda i: (0, 0))
            n16 = pl.BlockSpec((BN, 16), lambda i: (i, 0))
            f0, f1, elp0, elp1, erp0, erp1, mel0, mel1, mer0, mer1 = pl.pallas_call(
                _gat_head_body,
                grid=(NB,),
                in_specs=[row_o] + agg_specs + [b1, _full_spec((D, 2 * D)),
                                               b1, b1, b1, b1,
                                               _full_spec((D, 16))],
                out_specs=[row_o, row_o, n16, n16, n16, n16, m8, m8, m8, m8],
                out_shape=[jax.ShapeDtypeStruct((N, D), F32)] * 2
                + [jax.ShapeDtypeStruct((N, 16), F32)] * 4
                + [jax.ShapeDtypeStruct((8, D), F32)] * 4,
            )(hs, *agg_in, bsum[l], gwpack, al0, al1, ar0, ar1, gmat)

    mel0v = mel0[0, :16]
    mel1v = mel1[0, :16]
    mer0v = mer0[0, :16]
    mer1v = mer1[0, :16]

    nump0, denp0 = gat(f0, elp0, erp0, mel0v, mer0v, src0, dst0, z128, z16)
    nump1, denp1 = gat(f1, elp1, erp1, mel1v, mer1v, src1, dst1, z128, z16)

    out = pl.pallas_call(
        _final_body,
        grid=(NB,),
        in_specs=[_row_spec(0), _row_spec(1), _row_spec(0), _row_spec(1),
                  _deg_spec(0), _deg_spec(1), _deg_spec(0), _deg_spec(1),
                  _full_spec((16, D)), _full_spec((D, D)), b1],
        out_specs=row_o,
        out_shape=jax.ShapeDtypeStruct((N, D), F32),
    )(nump0, nump0, nump1, nump1, denp0, denp0, denp1, denp1,
      gmat_t, lin_W, linb2)
    return out


# speedup vs baseline: 16.1051x; 16.1051x over previous
import jax
import jax.numpy as jnp
from jax import lax
from jax.experimental import pallas as pl
from jax.experimental.pallas import tpu as pltpu
from jax.experimental.pallas import tpu_sc as plsc

N = 10000
E = 320000
D = 128
H = 4
DH = 32
NLAYER = 4
NC = 2
NS = 16
NW = NC * NS
EW = E // NW
CH = 80
NCH = EW // CH
NP = 10240
RPT = NP // NS
BN = 1000
NB = N // BN

F32 = jnp.float32
I32 = jnp.int32

_SC_PARAMS = dict(
    compiler_params=pltpu.CompilerParams(use_tc_tiling_on_sc=False),
    mesh=plsc.VectorSubcoreMesh(core_axis_name="c", subcore_axis_name="s",
                                num_cores=NC, num_subcores=NS),
)


def _mm0_body(x_ref, w_ref, hs_ref, hn0_ref, hn1_ref):
    acc = jnp.dot(x_ref[...], w_ref[...], preferred_element_type=F32)
    hs_ref[...] = acc[:, :D]
    hn0_ref[...] = acc[:, D:2 * D]
    hn1_ref[...] = acc[:, 2 * D:]


def _combine(hs_ref, a0_ref, a1_ref, dg_ref, b_ref):
    agg0 = a0_ref[0] + a0_ref[1]
    agg1 = a1_ref[0] + a1_ref[1]
    deg0 = jnp.maximum(dg_ref[0, 0, :, 0:1] + dg_ref[0, 1, :, 0:1], 1.0)
    deg1 = jnp.maximum(dg_ref[1, 0, :, 0:1] + dg_ref[1, 1, :, 0:1], 1.0)
    h = 0.5 * (hs_ref[...] + agg0 / deg0 + agg1 / deg1 + b_ref[...])
    return jnp.maximum(h, 0.01 * h)


def _layer_body(hs_ref, a0_ref, a1_ref, dg_ref, b_ref, w_ref,
                hs_o, hn0_o, hn1_o):
    h = _combine(hs_ref, a0_ref, a1_ref, dg_ref, b_ref)
    acc = jnp.dot(h, w_ref[...], preferred_element_type=F32)
    hs_o[...] = acc[:, :D]
    hn0_o[...] = acc[:, D:2 * D]
    hn1_o[...] = acc[:, 2 * D:]


def _gat_head_body(hs_ref, a0_ref, a1_ref, dg_ref, b_ref,
                   w_ref, al0_ref, al1_ref, ar0_ref, ar1_ref, g_ref,
                   f0_o, f1_o, elp0_o, elp1_o, erp0_o, erp1_o,
                   mel0_o, mel1_o, mer0_o, mer1_o):
    h = _combine(hs_ref, a0_ref, a1_ref, dg_ref, b_ref)
    acc = jnp.dot(h, w_ref[...], preferred_element_type=F32)
    f0 = acc[:, :D]
    f1 = acc[:, D:]
    f0_o[...] = f0
    f1_o[...] = f1
    g = g_ref[...]
    elp0 = jnp.dot(f0 * al0_ref[...], g, preferred_element_type=F32)
    elp1 = jnp.dot(f1 * al1_ref[...], g, preferred_element_type=F32)
    erp0 = jnp.dot(f0 * ar0_ref[...], g, preferred_element_type=F32)
    erp1 = jnp.dot(f1 * ar1_ref[...], g, preferred_element_type=F32)
    elp0_o[...] = elp0
    elp1_o[...] = elp1
    erp0_o[...] = erp0
    erp1_o[...] = erp1
    i = pl.program_id(0)

    @pl.when(i == 0)
    def _():
        mel0_o[...] = jnp.full_like(mel0_o, -1e30)
        mel1_o[...] = jnp.full_like(mel1_o, -1e30)
        mer0_o[...] = jnp.full_like(mer0_o, -1e30)
        mer1_o[...] = jnp.full_like(mer1_o, -1e30)

    mel0_o[...] = jnp.maximum(mel0_o[...], jnp.max(elp0))
    mel1_o[...] = jnp.maximum(mel1_o[...], jnp.max(elp1))
    mer0_o[...] = jnp.maximum(mer0_o[...], jnp.max(erp0))
    mer1_o[...] = jnp.maximum(mer1_o[...], jnp.max(erp1))


def _final_body(n0_ref, n1_ref, dn0_ref, dn1_ref, gt_ref, w_ref, b_ref,
                o_ref):
    gt = gt_ref[...]
    den0 = jnp.dot(dn0_ref[0] + dn0_ref[1], gt, preferred_element_type=F32)
    den1 = jnp.dot(dn1_ref[0] + dn1_ref[1], gt, preferred_element_type=F32)
    g0 = (n0_ref[0] + n0_ref[1]) / jnp.maximum(den0, 1e-30)
    g1 = (n1_ref[0] + n1_ref[1]) / jnp.maximum(den1, 1e-30)
    g = 0.5 * (g0 + g1)
    o_ref[...] = jnp.dot(g, w_ref[...], preferred_element_type=F32) + b_ref[...]


def _full_spec(shape):
    return pl.BlockSpec(shape, lambda i: tuple(0 for _ in shape))


def _spmm_body(hn_ref, src_ref, dst_ref, zer_ref, out_ref,
               acc, sidx, dbuf, rows, sem):
    c = lax.axis_index("c")
    s = lax.axis_index("s")
    wid = c * NS + s
    pltpu.sync_copy(zer_ref, acc.at[pl.ds(s * RPT, RPT)])
    base = wid * EW
    pltpu.sync_copy(src_ref.at[pl.ds(base, EW)], sidx)
    plsc.subcore_barrier()

    @pl.loop(0, NCH)
    def _(j):
        off = base + j * CH
        pltpu.sync_copy(dst_ref.at[pl.ds(off, CH)], dbuf)
        pltpu.async_copy(hn_ref.at[sidx.at[pl.ds(j * CH, CH)]], rows, sem).wait()
        pltpu.sync_copy(rows, acc.at[dbuf], add=True)

    plsc.subcore_barrier()
    pltpu.sync_copy(acc.at[pl.ds(s * RPT, RPT)],
                    out_ref.at[c, pl.ds(s * RPT, RPT)])


def _deg_body(dst0_ref, dst1_ref, zer_ref, ones_ref, out_ref,
              acc0, acc1, dbuf, ones_v):
    c = lax.axis_index("c")
    s = lax.axis_index("s")
    wid = c * NS + s
    pltpu.sync_copy(zer_ref, acc0.at[pl.ds(s * RPT, RPT)])
    pltpu.sync_copy(zer_ref, acc1.at[pl.ds(s * RPT, RPT)])
    pltpu.sync_copy(ones_ref, ones_v)
    plsc.subcore_barrier()
    base = wid * EW

    @pl.loop(0, NCH)
    def _(j):
        off = base + j * CH
        pltpu.sync_copy(dst0_ref.at[pl.ds(off, CH)], dbuf)
        pltpu.sync_copy(ones_v, acc0.at[dbuf], add=True)
        pltpu.sync_copy(dst1_ref.at[pl.ds(off, CH)], dbuf)
        pltpu.sync_copy(ones_v, acc1.at[dbuf], add=True)

    plsc.subcore_barrier()
    pltpu.sync_copy(acc0.at[pl.ds(s * RPT, RPT)],
                    out_ref.at[0, c, pl.ds(s * RPT, RPT)])
    pltpu.sync_copy(acc1.at[pl.ds(s * RPT, RPT)],
                    out_ref.at[1, c, pl.ds(s * RPT, RPT)])


def _gat_body(feat_ref, elp_ref, erp_ref, mel_ref, mer_ref, src_ref, dst_ref,
              z128_ref, z16_ref, num_ref, den_ref,
              accn, accd, sbuf, dbuf, fbuf, elb, erb, mv, sem):
    c = lax.axis_index("c")
    s = lax.axis_index("s")
    wid = c * NS + s
    pltpu.sync_copy(z128_ref, accn.at[pl.ds(s * RPT, RPT)])
    pltpu.sync_copy(z16_ref, accd.at[pl.ds(s * RPT, RPT)])
    pltpu.sync_copy(mel_ref, mv)
    pltpu.sync_copy(mer_ref, elb.at[0])
    plsc.subcore_barrier()
    mv[...] = mv[...] + elb[0, :]
    lane = lax.broadcasted_iota(I32, (16,), 0)
    base = wid * EW

    @pl.loop(0, NCH)
    def _(j):
        off = base + j * CH
        pltpu.sync_copy(src_ref.at[pl.ds(off, CH)], sbuf)
        pltpu.sync_copy(dst_ref.at[pl.ds(off, CH)], dbuf)
        pltpu.async_copy(elp_ref.at[sbuf], elb, sem).wait()
        pltpu.async_copy(erp_ref.at[dbuf], erb, sem).wait()
        pltpu.async_copy(feat_ref.at[sbuf], fbuf, sem).wait()
        m = mv[...]

        @pl.loop(0, CH)
        def _(i):
            e = elb[i, :] + erb[i, :]
            e = jnp.maximum(e, 0.2 * e)
            ee = jnp.exp(e - m)
            ee = jnp.where(lane < H, ee, 0.0)
            elb[i, :] = ee
            for hh in range(H):
                eh = ee[hh]
                for q in range(2):
                    col = hh * DH + q * 16
                    fbuf[i, pl.ds(col, 16)] = fbuf[i, pl.ds(col, 16)] * eh

        pltpu.sync_copy(fbuf, accn.at[dbuf], add=True)
        pltpu.sync_copy(elb, accd.at[dbuf], add=True)

    plsc.subcore_barrier()
    pltpu.sync_copy(accn.at[pl.ds(s * RPT, RPT)],
                    num_ref.at[c, pl.ds(s * RPT, RPT)])
    pltpu.sync_copy(accd.at[pl.ds(s * RPT, RPT)],
                    den_ref.at[c, pl.ds(s * RPT, RPT)])


def _make_spmm():
    return pl.kernel(
        _spmm_body,
        out_type=jax.ShapeDtypeStruct((NC, NP, D), F32),
        scratch_types=[
            pltpu.VMEM_SHARED((NP, D), F32),
            pltpu.VMEM((EW,), I32),
            pltpu.VMEM((CH,), I32),
            pltpu.VMEM((CH, D), F32),
            pltpu.SemaphoreType.DMA,
        ],
        **_SC_PARAMS,
    )


def _make_deg():
    return pl.kernel(
        _deg_body,
        out_type=jax.ShapeDtypeStruct((2, NC, NP, 16), F32),
        scratch_types=[
            pltpu.VMEM_SHARED((NP, 16), F32),
            pltpu.VMEM_SHARED((NP, 16), F32),
            pltpu.VMEM((CH,), I32),
            pltpu.VMEM((CH, 16), F32),
        ],
        **_SC_PARAMS,
    )


def _make_gat():
    return pl.kernel(
        _gat_body,
        out_type=[jax.ShapeDtypeStruct((NC, NP, D), F32),
                  jax.ShapeDtypeStruct((NC, NP, 16), F32)],
        scratch_types=[
            pltpu.VMEM_SHARED((NP, D), F32),
            pltpu.VMEM_SHARED((NP, 16), F32),
            pltpu.VMEM((CH,), I32),
            pltpu.VMEM((CH,), I32),
            pltpu.VMEM((CH, D), F32),
            pltpu.VMEM((CH, 16), F32),
            pltpu.VMEM((CH, 16), F32),
            pltpu.VMEM((16,), F32),
            pltpu.SemaphoreType.DMA,
        ],
        **_SC_PARAMS,
    )


def kernel(x, edge_index_r0, edge_index_r1, sage_Wself, sage_Wneigh, sage_b,
           gat_W, gat_attn_l, gat_attn_r, lin_W, lin_b):
    src0, dst0 = edge_index_r0[0], edge_index_r0[1]
    src1, dst1 = edge_index_r1[0], edge_index_r1[1]

    wpack = [jnp.concatenate([sage_Wself[l, 0] + sage_Wself[l, 1],
                              sage_Wneigh[l, 0], sage_Wneigh[l, 1]], axis=1)
             for l in range(NLAYER)]
    bsum = [(sage_b[l, 0] + sage_b[l, 1]).reshape(1, D) for l in range(NLAYER)]
    gwpack = jnp.concatenate([gat_W[0], gat_W[1]], axis=1)
    al0 = gat_attn_l[0].reshape(1, D)
    al1 = gat_attn_l[1].reshape(1, D)
    ar0 = gat_attn_r[0].reshape(1, D)
    ar1 = gat_attn_r[1].reshape(1, D)
    gmat = (jnp.arange(16)[None, :] == (jnp.arange(D)[:, None] // DH)).astype(F32)
    gmat_t = gmat.T
    linb2 = lin_b.reshape(1, D)
    z128 = jnp.zeros((RPT, D), F32)
    z16 = jnp.zeros((RPT, 16), F32)
    ones16 = jnp.ones((CH, 16), F32)

    spmm = _make_spmm()
    gat = _make_gat()

    degp = _make_deg()(dst0, dst1, z16, ones16)

    row_o = pl.BlockSpec((BN, D), lambda i: (i, 0))
    part = pl.BlockSpec((NC, BN, D), lambda i: (0, i, 0))
    part16 = pl.BlockSpec((NC, BN, 16), lambda i: (0, i, 0))
    degspec = pl.BlockSpec((2, NC, BN, 16), lambda i: (0, 0, i, 0))
    w384 = _full_spec((D, 3 * D))
    b1 = _full_spec((1, D))

    hs, hn0, hn1 = pl.pallas_call(
        _mm0_body,
        grid=(NB,),
        in_specs=[row_o, w384],
        out_specs=[row_o, row_o, row_o],
        out_shape=[jax.ShapeDtypeStruct((N, D), F32)] * 3,
    )(x, wpack[0])

    f0 = f1 = elp0 = elp1 = erp0 = erp1 = None
    mel0 = mel1 = mer0 = mer1 = None
    for l in range(NLAYER):
        agg0 = spmm(hn0, src0, dst0, z128)
        agg1 = spmm(hn1, src1, dst1, z128)
        if l < NLAYER - 1:
            hs, hn0, hn1 = pl.pallas_call(
                _layer_body,
                grid=(NB,),
                in_specs=[row_o, part, part, degspec, b1, w384],
                out_specs=[row_o, row_o, row_o],
                out_shape=[jax.ShapeDtypeStruct((N, D), F32)] * 3,
            )(hs, agg0, agg1, degp, bsum[l], wpack[l + 1])
        else:
            m8 = pl.BlockSpec((8, D), lambda i: (0, 0))
            n16 = pl.BlockSpec((BN, 16), lambda i: (i, 0))
            f0, f1, elp0, elp1, erp0, erp1, mel0, mel1, mer0, mer1 = pl.pallas_call(
                _gat_head_body,
                grid=(NB,),
                in_specs=[row_o, part, part, degspec, b1,
                          _full_spec((D, 2 * D)), b1, b1, b1, b1,
                          _full_spec((D, 16))],
                out_specs=[row_o, row_o, n16, n16, n16, n16, m8, m8, m8, m8],
                out_shape=[jax.ShapeDtypeStruct((N, D), F32)] * 2
                + [jax.ShapeDtypeStruct((N, 16), F32)] * 4
                + [jax.ShapeDtypeStruct((8, D), F32)] * 4,
            )(hs, agg0, agg1, degp, bsum[l], gwpack, al0, al1, ar0, ar1, gmat)

    mel0v = mel0[0, :16]
    mel1v = mel1[0, :16]
    mer0v = mer0[0, :16]
    mer1v = mer1[0, :16]

    num0, den0 = gat(f0, elp0, erp0, mel0v, mer0v, src0, dst0, z128, z16)
    num1, den1 = gat(f1, elp1, erp1, mel1v, mer1v, src1, dst1, z128, z16)

    out = pl.pallas_call(
        _final_body,
        grid=(NB,),
        in_specs=[part, part, part16, part16,
                  _full_spec((16, D)), _full_spec((D, D)), b1],
        out_specs=row_o,
        out_shape=jax.ShapeDtypeStruct((N, D), F32),
    )(num0, num1, den0, den1, gmat_t, lin_W, linb2)
    return out
